# baseline (speedup 1.0000x reference)
import functools

import jax
import jax.numpy as jnp
from jax import lax
from jax.experimental import pallas as pl
from jax.experimental.pallas import tpu as pltpu
from jax.experimental.pallas import tpu_sc as plsc

B, N, D, K = 16, 4096, 512, 2048
PD = 3
NC, NS = 2, 16
NW = NC * NS
RPW = (B * K) // NW
CH = 64
NBUF = 3
NCH = RPW // CH
L = 16

_mesh = plsc.VectorSubcoreMesh(core_axis_name="c", subcore_axis_name="s")


@functools.partial(
    pl.kernel,
    out_type=(
        jax.ShapeDtypeStruct((B * K, D), jnp.float32),
        jax.ShapeDtypeStruct((B * K * PD,), jnp.float32),
    ),
    mesh=_mesh,
    compiler_params=pltpu.CompilerParams(
        needs_layout_passes=False, use_tc_tiling_on_sc=False),
    scratch_types=[
        pltpu.VMEM((RPW,), jnp.int32),
        pltpu.VMEM((RPW,), jnp.int32),
        pltpu.VMEM((N * PD,), jnp.float32),
        pltpu.VMEM((NBUF, CH, D), jnp.float32),
        pltpu.VMEM((NBUF * CH * PD,), jnp.float32),
        pltpu.SemaphoreType.DMA,
        pltpu.SemaphoreType.DMA,
        pltpu.SemaphoreType.DMA,
        pltpu.SemaphoreType.DMA,
    ],
)
def _node_drop_sc(x_hbm, pos_hbm, idx_hbm, out_x, out_p,
                  idx_v, pidx_v, posg_v, xbuf, pbuf, sgx, sgp, swx, swp):
    wid = lax.axis_index("s") * NC + lax.axis_index("c")
    base = wid * RPW
    graph = wid // (K // RPW)

    pltpu.sync_copy(idx_hbm.at[pl.ds(base, RPW)], idx_v)
    posg_dma = pltpu.async_copy(
        pos_hbm.at[pl.ds(graph * (N * PD), N * PD)], posg_v, sgp)
    off = graph * N
    for j in range(RPW // L):
        sl = pl.ds(j * L, L)
        li = idx_v[sl]
        idx_v[sl] = li + off
        pidx_v[sl] = li * PD
    posg_dma.wait()

    lane = jnp.arange(L, dtype=jnp.int32)
    gx, wx, wp = {}, {}, {}

    def stage_chunk(c):
        buf = c % NBUF
        gx[c] = pltpu.async_copy(
            x_hbm.at[idx_v.at[pl.ds(c * CH, CH)]], xbuf.at[buf], sgx)
        pb_off = buf * (CH * PD)
        for g in range(CH // L):
            pb = pidx_v[pl.ds(c * CH + g * L, L)]
            dst0 = pb_off + (g * L) * PD + lane * PD
            for j in range(PD):
                v = plsc.load_gather(posg_v, [pb + j])
                plsc.store_scatter(pbuf, [dst0 + j], v)

    for c in range(min(NBUF, NCH)):
        stage_chunk(c)

    for c in range(NCH):
        gx[c].wait()
        buf = c % NBUF
        wx[c] = pltpu.async_copy(
            xbuf.at[buf], out_x.at[pl.ds(base + c * CH, CH)], swx)
        wp[c] = pltpu.async_copy(
            pbuf.at[pl.ds(buf * (CH * PD), CH * PD)],
            out_p.at[pl.ds((base + c * CH) * PD, CH * PD)], swp)
        n = c + NBUF
        if n < NCH:
            wx[c].wait()
            wp[c].wait()
            stage_chunk(n)

    for c in range(max(0, NCH - NBUF), NCH):
        wx[c].wait()
        wp[c].wait()


def kernel(x, pos, mask_idx):
    xf = x.reshape(B * N, D)
    pf = pos.reshape(B * N * PD)
    idxf = mask_idx.reshape(B * K).astype(jnp.int32)
    ox, op = _node_drop_sc(xf, pf, idxf)
    return ox.reshape(B, K, D), op.reshape(B, K, PD)


# speedup vs baseline: 2.0036x; 2.0036x over previous
import functools

import jax
import jax.numpy as jnp
from jax import lax
from jax.experimental import pallas as pl
from jax.experimental.pallas import tpu as pltpu
from jax.experimental.pallas import tpu_sc as plsc

B, N, D, K = 16, 4096, 512, 2048
PD = 3
NC, NS = 2, 16
NW = NC * NS
RPW = (B * K) // NW
CH = 64
NBUF = 3
NCH = RPW // CH
L = 16

_mesh = plsc.VectorSubcoreMesh(core_axis_name="c", subcore_axis_name="s")


@functools.partial(
    pl.kernel,
    out_type=(
        jax.ShapeDtypeStruct((B * K, D), jnp.float32),
        jax.ShapeDtypeStruct((B * K * PD,), jnp.float32),
    ),
    mesh=_mesh,
    compiler_params=pltpu.CompilerParams(needs_layout_passes=False),
    scratch_types=[
        pltpu.VMEM((RPW,), jnp.int32),
        pltpu.VMEM((RPW,), jnp.int32),
        pltpu.VMEM((N * PD,), jnp.float32),
        pltpu.VMEM((NBUF, CH, D), jnp.float32),
        pltpu.VMEM((NBUF * CH * PD,), jnp.float32),
        pltpu.SemaphoreType.DMA,
        pltpu.SemaphoreType.DMA,
        pltpu.SemaphoreType.DMA,
        pltpu.SemaphoreType.DMA,
    ],
)
def _node_drop_sc(x_hbm, pos_hbm, idx_hbm, out_x, out_p,
                  idx_v, pidx_v, posg_v, xbuf, pbuf, sgx, sgp, swx, swp):
    wid = lax.axis_index("s") * NC + lax.axis_index("c")
    base = wid * RPW
    graph = wid // (K // RPW)

    pltpu.sync_copy(idx_hbm.at[pl.ds(base, RPW)], idx_v)
    posg_dma = pltpu.async_copy(
        pos_hbm.at[pl.ds(graph * (N * PD), N * PD)], posg_v, sgp)
    off = graph * N
    for j in range(RPW // L):
        sl = pl.ds(j * L, L)
        li = idx_v[sl]
        idx_v[sl] = li + off
        pidx_v[sl] = li * PD
    posg_dma.wait()

    lane = jnp.arange(L, dtype=jnp.int32)
    gx, wx, wp = {}, {}, {}

    def stage_chunk(c):
        buf = c % NBUF
        gx[c] = pltpu.async_copy(
            x_hbm.at[idx_v.at[pl.ds(c * CH, CH)]], xbuf.at[buf], sgx)
        pb_off = buf * (CH * PD)
        for g in range(CH // L):
            pb = pidx_v[pl.ds(c * CH + g * L, L)]
            dst0 = pb_off + (g * L) * PD + lane * PD
            for j in range(PD):
                v = plsc.load_gather(posg_v, [pb + j])
                plsc.store_scatter(pbuf, [dst0 + j], v)

    for c in range(min(NBUF, NCH)):
        stage_chunk(c)

    for c in range(NCH):
        gx[c].wait()
        buf = c % NBUF
        wx[c] = pltpu.async_copy(
            xbuf.at[buf], out_x.at[pl.ds(base + c * CH, CH)], swx)
        wp[c] = pltpu.async_copy(
            pbuf.at[pl.ds(buf * (CH * PD), CH * PD)],
            out_p.at[pl.ds((base + c * CH) * PD, CH * PD)], swp)
        n = c + NBUF
        if n < NCH:
            wx[c].wait()
            wp[c].wait()
            stage_chunk(n)

    for c in range(max(0, NCH - NBUF), NCH):
        wx[c].wait()
        wp[c].wait()


def kernel(x, pos, mask_idx):
    xf = x.reshape(B * N, D)
    pf = pos.reshape(B * N * PD)
    idxf = mask_idx.reshape(B * K).astype(jnp.int32)
    ox, op = _node_drop_sc(xf, pf, idxf)
    return ox.reshape(B, K, D), op.reshape(B, K, PD)


# speedup vs baseline: 3.1735x; 1.5839x over previous
import functools

import jax
import jax.numpy as jnp
from jax import lax
from jax.experimental import pallas as pl
from jax.experimental.pallas import tpu as pltpu
from jax.experimental.pallas import tpu_sc as plsc

B, N, D, K = 16, 4096, 512, 2048
PD = 3
NC, NS = 2, 16
NW = NC * NS
WPG = NW // B
RPW = (B * K) // NW
CH = 64
NBUF = 2
NCH = RPW // CH
L = 16

_mesh = plsc.VectorSubcoreMesh(core_axis_name="c", subcore_axis_name="s")


@functools.partial(
    pl.kernel,
    out_type=(
        jax.ShapeDtypeStruct((B, K, D), jnp.float32),
        jax.ShapeDtypeStruct((B, K, PD), jnp.float32),
    ),
    mesh=_mesh,
    compiler_params=pltpu.CompilerParams(needs_layout_passes=False),
    scratch_types=[
        pltpu.VMEM((RPW,), jnp.int32),
        pltpu.VMEM((PD * N,), jnp.float32),
        pltpu.VMEM((NBUF, CH, D), jnp.float32),
        pltpu.VMEM((NBUF, CH, PD), jnp.float32),
        pltpu.SemaphoreType.DMA,
        pltpu.SemaphoreType.DMA,
        pltpu.SemaphoreType.DMA,
        pltpu.SemaphoreType.DMA,
    ],
)
def _node_drop_sc(x_hbm, post_hbm, idx_hbm, out_x, out_p,
                  idx_v, posg_v, xbuf, pbuf, sgx, sgp, swx, swp):
    wid = lax.axis_index("s") * NC + lax.axis_index("c")
    graph = wid // WPG
    lbase = (wid % WPG) * RPW

    pltpu.sync_copy(idx_hbm.at[graph, pl.ds(lbase, RPW)], idx_v)
    posg_dma = pltpu.async_copy(post_hbm.at[graph], posg_v, sgp)
    posg_dma.wait()

    lane = jnp.arange(L, dtype=jnp.int32)
    gx, wx, wp = {}, {}, {}

    def stage_chunk(c):
        buf = c % NBUF
        gx[c] = pltpu.async_copy(
            x_hbm.at[graph].at[idx_v.at[pl.ds(c * CH, CH)]],
            xbuf.at[buf], sgx)
        for g in range(CH // L):
            li = idx_v[pl.ds(c * CH + g * L, L)]
            row = g * L + lane
            for j in range(PD):
                v = plsc.load_gather(posg_v, [li + (j * N)])
                plsc.store_scatter(
                    pbuf.at[buf], [row, jnp.full((L,), j, jnp.int32)], v)

    for c in range(min(NBUF, NCH)):
        stage_chunk(c)

    for c in range(NCH):
        gx[c].wait()
        buf = c % NBUF
        dst = pl.ds(lbase + c * CH, CH)
        wx[c] = pltpu.async_copy(xbuf.at[buf], out_x.at[graph, dst], swx)
        wp[c] = pltpu.async_copy(pbuf.at[buf], out_p.at[graph, dst], swp)
        n = c + NBUF
        if n < NCH:
            wx[c].wait()
            wp[c].wait()
            stage_chunk(n)

    for c in range(max(0, NCH - NBUF), NCH):
        wx[c].wait()
        wp[c].wait()


def kernel(x, pos, mask_idx):
    pos_t = jnp.swapaxes(pos, 1, 2).reshape(B, PD * N)
    return _node_drop_sc(x, pos_t, mask_idx.astype(jnp.int32))


# speedup vs baseline: 3.8736x; 1.2206x over previous
import functools

import jax
import jax.numpy as jnp
from jax import lax
from jax.experimental import pallas as pl
from jax.experimental.pallas import tpu as pltpu
from jax.experimental.pallas import tpu_sc as plsc

B, N, D, K = 16, 4096, 512, 2048
PD = 3
NC, NS = 2, 16
NW = NC * NS
WPG = NW // B
RPW = (B * K) // NW
CH = 64
NBUF = 3
NCH = RPW // CH
L = 16

_mesh = plsc.VectorSubcoreMesh(core_axis_name="c", subcore_axis_name="s")


@functools.partial(
    pl.kernel,
    out_type=(
        jax.ShapeDtypeStruct((B, K, D), jnp.float32),
        jax.ShapeDtypeStruct((B, PD, K), jnp.float32),
    ),
    mesh=_mesh,
    compiler_params=pltpu.CompilerParams(needs_layout_passes=False),
    scratch_types=[
        pltpu.VMEM((RPW,), jnp.int32),
        pltpu.VMEM((PD * N,), jnp.float32),
        pltpu.VMEM((NBUF, CH, D), jnp.float32),
        pltpu.VMEM((PD, RPW), jnp.float32),
        pltpu.SemaphoreType.DMA,
        pltpu.SemaphoreType.DMA,
        pltpu.SemaphoreType.DMA,
        pltpu.SemaphoreType.DMA,
    ],
)
def _node_drop_sc(x_hbm, post_hbm, idx_hbm, out_x, out_pt,
                  idx_v, posg_v, xbuf, pbuf, sgx, sgp, swx, swp):
    wid = lax.axis_index("s") * NC + lax.axis_index("c")
    graph = wid // WPG
    lbase = (wid % WPG) * RPW

    pltpu.sync_copy(idx_hbm.at[graph, pl.ds(lbase, RPW)], idx_v)
    posg_dma = pltpu.async_copy(post_hbm.at[graph], posg_v, sgp)

    gx, wx = {}, {}
    x_graph = x_hbm.at[graph]

    def start_gather(c):
        gx[c] = pltpu.async_copy(
            x_graph.at[idx_v.at[pl.ds(c * CH, CH)]], xbuf.at[c % NBUF], sgx)

    for c in range(min(NBUF, NCH)):
        start_gather(c)

    posg_dma.wait()
    for g in range(RPW // L):
        li = idx_v[pl.ds(g * L, L)]
        for j in range(PD):
            pbuf[j, pl.ds(g * L, L)] = plsc.load_gather(posg_v, [li + (j * N)])
    wpos = pltpu.async_copy(
        pbuf, out_pt.at[graph, :, pl.ds(lbase, RPW)], swp)

    for c in range(NCH):
        gx[c].wait()
        wx[c] = pltpu.async_copy(
            xbuf.at[c % NBUF], out_x.at[graph, pl.ds(lbase + c * CH, CH)], swx)
        n = c + NBUF
        if n < NCH:
            wx[c].wait()
            start_gather(n)

    for c in range(max(0, NCH - NBUF), NCH):
        wx[c].wait()
    wpos.wait()


def kernel(x, pos, mask_idx):
    pos_t = jnp.swapaxes(pos, 1, 2).reshape(B, PD * N)
    ox, opt = _node_drop_sc(x, pos_t, mask_idx.astype(jnp.int32))
    return ox, jnp.swapaxes(opt, 1, 2)
